# Initial kernel scaffold; baseline (speedup 1.0000x reference)
import jax
import jax.numpy as jnp
from jax.experimental import pallas as pl


def kernel(E1, E2):
    raise NotImplementedError("write your pallas kernel here")


# speedup vs baseline: 15.2118x; 15.2118x over previous
import functools

import jax
import jax.numpy as jnp
from jax.experimental import pallas as pl

_TOPK = 128
_NEG_CAP = -20.0


def _body(topk, e1_ref, e2_ref, a_ref, proxy_ref, logits_ref):
    e1 = e1_ref[...]
    e2 = e2_ref[...]
    v = jax.lax.dot_general(
        e1, e2, (((1,), (1,)), ((), ())), preferred_element_type=jnp.float32
    )
    v = jnp.maximum(v, 0.0)

    b_rows, n = v.shape
    bits = jax.lax.bitcast_convert_type(v, jnp.int32) & jnp.int32(0x7FFFFFFF)

    t = jnp.zeros((b_rows, 1), jnp.int32)
    for b in range(30, -1, -1):
        cand = t | jnp.int32(1 << b)
        cnt = jnp.sum((bits >= cand).astype(jnp.int32), axis=1, keepdims=True)
        t = jnp.where(cnt >= topk, cand, t)

    gt = bits > t
    eq = bits == t
    c_gt = jnp.sum(gt.astype(jnp.int32), axis=1, keepdims=True)
    need = topk - c_gt

    col = jax.lax.broadcasted_iota(jnp.int32, (b_rows, n), 1)
    cut = jnp.zeros((b_rows, 1), jnp.int32)
    for b in range(11, -1, -1):
        cand = cut | jnp.int32(1 << b)
        cnt = jnp.sum((eq & (col < cand)).astype(jnp.int32), axis=1, keepdims=True)
        cut = jnp.where(cnt < need, cand, cut)

    sel = gt | (eq & (col <= cut))

    m = jnp.max(v, axis=1, keepdims=True)
    ex = jnp.where(sel, jnp.exp(v - m), 0.0)
    s = jnp.sum(ex, axis=1, keepdims=True)
    a_ref[...] = ex / s
    proxy_ref[...] = jax.nn.sigmoid(jnp.where(sel, v, _NEG_CAP))
    logits_ref[...] = jnp.where(sel, v, -jnp.inf)


def kernel(E1, E2):
    n, emb = E1.shape
    block = 256
    grid = (n // block,)
    out = pl.pallas_call(
        functools.partial(_body, _TOPK),
        grid=grid,
        in_specs=[
            pl.BlockSpec((block, emb), lambda i: (i, 0)),
            pl.BlockSpec((n, emb), lambda i: (0, 0)),
        ],
        out_specs=[
            pl.BlockSpec((block, n), lambda i: (i, 0)),
            pl.BlockSpec((block, n), lambda i: (i, 0)),
            pl.BlockSpec((block, n), lambda i: (i, 0)),
        ],
        out_shape=[
            jax.ShapeDtypeStruct((n, n), jnp.float32),
            jax.ShapeDtypeStruct((n, n), jnp.float32),
            jax.ShapeDtypeStruct((n, n), jnp.float32),
        ],
    )(E1, E2)
    return tuple(out)


# speedup vs baseline: 19.8125x; 1.3024x over previous
import functools

import jax
import jax.numpy as jnp
from jax.experimental import pallas as pl
from jax.experimental.pallas import tpu as pltpu

_TOPK = 128
_NEG_CAP = -20.0


def _body(topk, e1_ref, e2_ref, a_ref, proxy_ref, logits_ref, cut_ref):
    e1 = e1_ref[...]
    e2 = e2_ref[...]
    v = jax.lax.dot_general(
        e1, e2, (((1,), (1,)), ((), ())), preferred_element_type=jnp.float32
    )
    v = jnp.maximum(v, 0.0)

    b_rows, n = v.shape
    bits = jax.lax.bitcast_convert_type(v, jnp.int32) & jnp.int32(0x7FFFFFFF)

    t = jnp.zeros((b_rows, 1), jnp.int32)
    for b in range(30, -1, -1):
        cand = t | jnp.int32(1 << b)
        cnt = jnp.sum((bits >= cand).astype(jnp.int32), axis=1, keepdims=True)
        t = jnp.where(cnt >= topk, cand, t)

    gt = bits > t
    eq = bits == t
    c_gt = jnp.sum(gt.astype(jnp.int32), axis=1, keepdims=True)
    c_eq = jnp.sum(eq.astype(jnp.int32), axis=1, keepdims=True)
    need = topk - c_gt

    col = jax.lax.broadcasted_iota(jnp.int32, (b_rows, n), 1)
    cut_ref[...] = jnp.full((b_rows, 1), n - 1, jnp.int32)

    @pl.when(jnp.logical_not(jnp.all(c_eq == need)))
    def _tie_break():
        cut = jnp.zeros((b_rows, 1), jnp.int32)
        for b in range(11, -1, -1):
            cand = cut | jnp.int32(1 << b)
            cnt = jnp.sum((eq & (col < cand)).astype(jnp.int32), axis=1,
                          keepdims=True)
            cut = jnp.where(cnt < need, cand, cut)
        cut_ref[...] = cut

    sel = gt | (eq & (col <= cut_ref[...]))

    m = jnp.max(v, axis=1, keepdims=True)
    ex = jnp.where(sel, jnp.exp(v - m), 0.0)
    s = jnp.sum(ex, axis=1, keepdims=True)
    a_ref[...] = ex / s
    proxy_ref[...] = jax.nn.sigmoid(jnp.where(sel, v, _NEG_CAP))
    logits_ref[...] = jnp.where(sel, v, -jnp.inf)


def kernel(E1, E2):
    n, emb = E1.shape
    block = 256
    grid = (n // block,)
    out = pl.pallas_call(
        functools.partial(_body, _TOPK),
        grid=grid,
        in_specs=[
            pl.BlockSpec((block, emb), lambda i: (i, 0)),
            pl.BlockSpec((n, emb), lambda i: (0, 0)),
        ],
        out_specs=[
            pl.BlockSpec((block, n), lambda i: (i, 0)),
            pl.BlockSpec((block, n), lambda i: (i, 0)),
            pl.BlockSpec((block, n), lambda i: (i, 0)),
        ],
        out_shape=[
            jax.ShapeDtypeStruct((n, n), jnp.float32),
            jax.ShapeDtypeStruct((n, n), jnp.float32),
            jax.ShapeDtypeStruct((n, n), jnp.float32),
        ],
        scratch_shapes=[pltpu.VMEM((block, 1), jnp.int32)],
    )(E1, E2)
    return tuple(out)
